# baseline (speedup 1.0000x reference)
import functools

import jax
import jax.numpy as jnp
from jax import lax
from jax.experimental import pallas as pl
from jax.experimental.pallas import tpu as pltpu
from jax.experimental.pallas import tpu_sc as plsc

VOCAB_N = 100000
D = 1024
BATCH_N = 4
SEQ_N = 2048
TOK_TOTAL = BATCH_N * SEQ_N
EPS_LN = 1e-5

NC = 2
NS = 16
L = 16
NW = NC * NS
SPW = SEQ_N // NW
CH = 32
NSC = SPW // CH
NJ = D // L


def _rsqrt_nr(x):
    i = plsc.bitcast(x, jnp.int32)
    i = jnp.int32(0x5F3759DF) - lax.shift_right_logical(i, 1)
    y = plsc.bitcast(i, jnp.float32)
    half = jnp.float32(0.5) * x
    for _ in range(3):
        y = y * (jnp.float32(1.5) - half * y * y)
    return y


def _emb_ln_body(xt_hbm, word_hbm, pos_hbm, tok_hbm, gamma_hbm, beta_hbm,
                 out_hbm, idx_v, wbuf, pbuf, tok_v, gam_v, bet_v, sem):
    wid = lax.axis_index("s") * NC + lax.axis_index("c")
    s0 = wid * SPW

    pltpu.sync_copy(xt_hbm.at[wid], idx_v)
    pltpu.sync_copy(tok_hbm.at[0], tok_v)
    pltpu.sync_copy(gamma_hbm, gam_v)
    pltpu.sync_copy(beta_hbm, bet_v)

    inv_d = jnp.float32(1.0 / D)

    for sc in range(NSC):
        pltpu.sync_copy(pos_hbm.at[pl.ds(s0 + sc * CH, CH)], pbuf)

        def batch_body(b, _, sc=sc):
            pltpu.async_copy(
                word_hbm.at[idx_v.at[b * NSC + sc]], wbuf, sem).wait()

            def row_body(r, _):
                acc = [jnp.zeros((L,), jnp.float32) for _ in range(4)]
                acc2 = [jnp.zeros((L,), jnp.float32) for _ in range(4)]
                for j in range(NJ):
                    w = wbuf[r, pl.ds(j * L, L)]
                    p = pbuf[r, pl.ds(j * L, L)]
                    t = tok_v[pl.ds(j * L, L)]
                    h = w + p + t
                    wbuf[r, pl.ds(j * L, L)] = h
                    k = j % 4
                    acc[k] = acc[k] + h
                    acc2[k] = acc2[k] + h * h
                s1 = jnp.sum((acc[0] + acc[1]) + (acc[2] + acc[3]))
                s2 = jnp.sum((acc2[0] + acc2[1]) + (acc2[2] + acc2[3]))
                mean = s1 * inv_d
                var = s2 * inv_d - mean * mean
                rstd = _rsqrt_nr(jnp.full((L,), var + EPS_LN, jnp.float32))
                mean_v = jnp.full((L,), mean, jnp.float32)
                for j in range(NJ):
                    h = wbuf[r, pl.ds(j * L, L)]
                    g = gam_v[pl.ds(j * L, L)]
                    bb = bet_v[pl.ds(j * L, L)]
                    wbuf[r, pl.ds(j * L, L)] = (h - mean_v) * rstd * g + bb

            lax.fori_loop(0, CH, row_body, None)

            base = pl.multiple_of(b * SEQ_N + s0 + sc * CH, CH)
            pltpu.sync_copy(wbuf, out_hbm.at[pl.ds(base, CH)])

        lax.fori_loop(0, BATCH_N, batch_body, None)


@jax.jit
def _emb_ln(xt, word_emb, pos_emb, tok_emb, gamma, beta):
    mesh = plsc.VectorSubcoreMesh(
        core_axis_name="c", subcore_axis_name="s",
        num_cores=NC, num_subcores=NS)
    return pl.kernel(
        _emb_ln_body,
        out_type=jax.ShapeDtypeStruct((TOK_TOTAL, D), jnp.float32),
        mesh=mesh,
        compiler_params=pltpu.CompilerParams(needs_layout_passes=False),
        scratch_types=[
            pltpu.VMEM((BATCH_N * NSC, CH), jnp.int32),
            pltpu.VMEM((CH, D), jnp.float32),
            pltpu.VMEM((CH, D), jnp.float32),
            pltpu.VMEM((D,), jnp.float32),
            pltpu.VMEM((D,), jnp.float32),
            pltpu.VMEM((D,), jnp.float32),
            pltpu.SemaphoreType.DMA,
        ],
    )(xt, word_emb, pos_emb, tok_emb, gamma, beta)


def kernel(x, word_emb, pos_emb, tok_emb, gamma, beta):
    xi = x.astype(jnp.int32)
    xt = xi.reshape(BATCH_N, NW, NSC, CH).transpose(1, 0, 2, 3)
    xt = xt.reshape(NW, BATCH_N * NSC, CH)
    out = _emb_ln(xt, word_emb, pos_emb, tok_emb, gamma, beta)
    return out.reshape(BATCH_N, SEQ_N, D)


# speedup vs baseline: 1.7228x; 1.7228x over previous
import jax
import jax.numpy as jnp
from jax import lax
from jax.experimental import pallas as pl
from jax.experimental.pallas import tpu as pltpu
from jax.experimental.pallas import tpu_sc as plsc

VOCAB_N = 100000
D = 1024
BATCH_N = 4
SEQ_N = 2048
TOK_TOTAL = BATCH_N * SEQ_N
EPS_LN = 1e-5

NC = 2
NS = 16
L = 16
NW = NC * NS
SPW = SEQ_N // NW
CH = 32
NSC = SPW // CH
NBLK = BATCH_N * NSC
NJ = D // L


def _rsqrt_nr(x):
    i = plsc.bitcast(x, jnp.int32)
    i = jnp.int32(0x5F3759DF) - lax.shift_right_logical(i, 1)
    y = plsc.bitcast(i, jnp.float32)
    half = jnp.float32(0.5) * x
    for _ in range(3):
        y = y * (jnp.float32(1.5) - half * y * y)
    return y


def _emb_ln_body(xt_hbm, word_hbm, pos_hbm, tok_hbm, gamma_hbm, beta_hbm,
                 out_hbm, idx_v, wbuf0, wbuf1, pbuf, tok_v, gam_v, bet_v,
                 sem0, sem1):
    wid = lax.axis_index("s") * NC + lax.axis_index("c")
    s0 = wid * SPW

    pltpu.sync_copy(xt_hbm.at[wid], idx_v)
    pltpu.sync_copy(tok_hbm.at[0], tok_v)
    pltpu.sync_copy(gamma_hbm, gam_v)
    pltpu.sync_copy(beta_hbm, bet_v)

    wbufs = (wbuf0, wbuf1)
    sems = (sem0, sem1)
    inv_d = jnp.float32(1.0 / D)

    def fire(k, d):
        pltpu.async_copy(word_hbm.at[idx_v.at[k]], wbufs[d], sems[d])

    fire(0, 0)

    @pl.loop(0, NBLK, step=2)
    def kloop(k0):
        for d in range(2):
            k = k0 + d
            sc = k // BATCH_N
            b = lax.rem(k, BATCH_N)

            @pl.when(b == 0)
            def _load_pos(sc=sc):
                pltpu.sync_copy(pos_hbm.at[pl.ds(s0 + sc * CH, CH)], pbuf)

                @plsc.parallel_loop(0, CH)
                def _fold_tok(r):
                    for j in range(NJ):
                        sl = pl.ds(j * L, L)
                        pbuf[r, sl] = pbuf[r, sl] + tok_v[sl]

            @pl.when(k < NBLK - 1)
            def _prefetch(k=k, d=d):
                fire(k + 1, 1 - d)

            pltpu.make_async_copy(
                word_hbm.at[idx_v.at[k]], wbufs[d], sems[d]).wait()
            wb = wbufs[d]

            @plsc.parallel_loop(0, CH)
            def _row(r, wb=wb):
                acc = [jnp.zeros((L,), jnp.float32) for _ in range(4)]
                acc2 = [jnp.zeros((L,), jnp.float32) for _ in range(4)]
                for j in range(NJ):
                    sl = pl.ds(j * L, L)
                    h = wb[r, sl] + pbuf[r, sl]
                    wb[r, sl] = h
                    m = j % 4
                    acc[m] = acc[m] + h
                    acc2[m] = acc2[m] + h * h
                s1 = jnp.sum((acc[0] + acc[1]) + (acc[2] + acc[3]))
                s2 = jnp.sum((acc2[0] + acc2[1]) + (acc2[2] + acc2[3]))
                mean = s1 * inv_d
                var = s2 * inv_d - mean * mean
                rstd = _rsqrt_nr(jnp.full((L,), var + EPS_LN, jnp.float32))
                mean_v = jnp.full((L,), mean, jnp.float32)
                for j in range(NJ):
                    sl = pl.ds(j * L, L)
                    wb[r, sl] = (wb[r, sl] - mean_v) * rstd * gam_v[sl] \
                        + bet_v[sl]

            base = pl.multiple_of(b * SEQ_N + s0 + sc * CH, CH)
            pltpu.sync_copy(wb, out_hbm.at[pl.ds(base, CH)])


@jax.jit
def _emb_ln(xt, word_emb, pos_emb, tok_emb, gamma, beta):
    mesh = plsc.VectorSubcoreMesh(
        core_axis_name="c", subcore_axis_name="s",
        num_cores=NC, num_subcores=NS)
    return pl.kernel(
        _emb_ln_body,
        out_type=jax.ShapeDtypeStruct((TOK_TOTAL, D), jnp.float32),
        mesh=mesh,
        compiler_params=pltpu.CompilerParams(needs_layout_passes=False),
        scratch_types=[
            pltpu.VMEM((NBLK, CH), jnp.int32),
            pltpu.VMEM((CH, D), jnp.float32),
            pltpu.VMEM((CH, D), jnp.float32),
            pltpu.VMEM((CH, D), jnp.float32),
            pltpu.VMEM((D,), jnp.float32),
            pltpu.VMEM((D,), jnp.float32),
            pltpu.VMEM((D,), jnp.float32),
            pltpu.SemaphoreType.DMA,
            pltpu.SemaphoreType.DMA,
        ],
    )(xt, word_emb, pos_emb, tok_emb, gamma, beta)


def kernel(x, word_emb, pos_emb, tok_emb, gamma, beta):
    xi = x.astype(jnp.int32)
    xt = xi.reshape(BATCH_N, NW, NSC, CH).transpose(1, 2, 0, 3)
    xt = xt.reshape(NW, NBLK, CH)
    out = _emb_ln(xt, word_emb, pos_emb, tok_emb, gamma, beta)
    return out.reshape(BATCH_N, SEQ_N, D)
